# baseline (speedup 1.0000x reference)
import jax
import jax.numpy as jnp
from jax.experimental import pallas as pl
from jax.experimental.pallas import tpu as pltpu

GAMMA_ = 2.0
ALPHA_ = 0.25

LB_ = 32768


def _focal_loss_kernel(x_ref, t_ref, w_ref, o_ref):
    x = x_ref[...]
    t = t_ref[...]
    w = w_ref[...]

    th = jnp.tanh(x * 0.5)
    sig = 0.5 + 0.5 * th
    q = sig - th
    bce = jnp.log(q) * -1.0 - x * t
    ts = t * sig
    pt = (sig - ts) + (t - ts)
    alpha_w = 0.75 - 0.5 * t
    o_ref[...] = (alpha_w * (pt * pt)) * (bce * w)


def kernel(input, target, weights, groups, epoch):
    B, A, C = input.shape
    xt = jnp.transpose(input, (2, 0, 1))
    tt = jnp.transpose(target, (2, 0, 1))
    wt = weights[None]

    grid = (A // LB_,)
    out = pl.pallas_call(
        _focal_loss_kernel,
        out_shape=jax.ShapeDtypeStruct((C, B, A), jnp.float32),
        grid=grid,
        in_specs=[
            pl.BlockSpec((C, B, LB_), lambda i: (0, 0, i)),
            pl.BlockSpec((C, B, LB_), lambda i: (0, 0, i)),
            pl.BlockSpec((1, B, LB_), lambda i: (0, 0, i)),
        ],
        out_specs=pl.BlockSpec((C, B, LB_), lambda i: (0, 0, i)),
        compiler_params=pltpu.CompilerParams(
            dimension_semantics=("parallel",),
        ),
    )(xt, tt, wt)
    return jnp.transpose(out, (1, 2, 0))


# speedup vs baseline: 1.0288x; 1.0288x over previous
import jax
import jax.numpy as jnp
from jax.experimental import pallas as pl
from jax.experimental.pallas import tpu as pltpu

GAMMA_ = 2.0
ALPHA_ = 0.25

LB_ = 65536


def _focal_loss_kernel(x_ref, t_ref, w_ref, o_ref):
    x = x_ref[...]
    t = t_ref[...]
    w = w_ref[...]

    th = jnp.tanh(x * 0.5)
    sig = 0.5 + 0.5 * th
    q = sig - th
    bce = jnp.log(q) * -1.0 - x * t
    ts = t * sig
    pt = (sig - ts) + (t - ts)
    alpha_w = 0.75 - 0.5 * t
    o_ref[...] = (alpha_w * (pt * pt)) * (bce * w)


def kernel(input, target, weights, groups, epoch):
    B, A, C = input.shape
    xt = jnp.transpose(input, (2, 0, 1))
    tt = jnp.transpose(target, (2, 0, 1))
    wt = weights[None]

    grid = (A // LB_,)
    out = pl.pallas_call(
        _focal_loss_kernel,
        out_shape=jax.ShapeDtypeStruct((C, B, A), jnp.float32),
        grid=grid,
        in_specs=[
            pl.BlockSpec((C, B, LB_), lambda i: (0, 0, i)),
            pl.BlockSpec((C, B, LB_), lambda i: (0, 0, i)),
            pl.BlockSpec((1, B, LB_), lambda i: (0, 0, i)),
        ],
        out_specs=pl.BlockSpec((C, B, LB_), lambda i: (0, 0, i)),
        compiler_params=pltpu.CompilerParams(
            dimension_semantics=("parallel",),
        ),
    )(xt, tt, wt)
    return jnp.transpose(out, (1, 2, 0))
